# baseline (speedup 1.0000x reference)
import jax
import jax.numpy as jnp
from jax.experimental import pallas as pl


def _mul_kernel(x1_ref, x2_ref, o_ref):
    o_ref[...] = x1_ref[...] * x2_ref[...]


def kernel(x1, x2):
    B, M, N = x1.shape
    R = B * M
    x1f = x1.reshape(R, N)
    x2f = x2.reshape(R, N)
    BS = 128
    out = pl.pallas_call(
        _mul_kernel,
        grid=(R // BS,),
        in_specs=[
            pl.BlockSpec((BS, N), lambda i: (i, 0)),
            pl.BlockSpec((BS, N), lambda i: (i, 0)),
        ],
        out_specs=pl.BlockSpec((BS, N), lambda i: (i, 0)),
        out_shape=jax.ShapeDtypeStruct((R, N), x1.dtype),
    )(x1f, x2f)
    return out.reshape(B, M, N)


# speedup vs baseline: 1.0305x; 1.0305x over previous
import jax
import jax.numpy as jnp
from jax.experimental import pallas as pl
from jax.experimental.pallas import tpu as pltpu


def _mul_kernel(x1_ref, x2_ref, o_ref):
    o_ref[...] = x1_ref[...] * x2_ref[...]


def kernel(x1, x2):
    B, M, N = x1.shape
    R = B * M
    x1f = x1.reshape(R, N)
    x2f = x2.reshape(R, N)
    BS = 256
    out = pl.pallas_call(
        _mul_kernel,
        grid=(R // BS,),
        compiler_params=pltpu.CompilerParams(
            dimension_semantics=("parallel",),
        ),
        in_specs=[
            pl.BlockSpec((BS, N), lambda i: (i, 0)),
            pl.BlockSpec((BS, N), lambda i: (i, 0)),
        ],
        out_specs=pl.BlockSpec((BS, N), lambda i: (i, 0)),
        out_shape=jax.ShapeDtypeStruct((R, N), x1.dtype),
    )(x1f, x2f)
    return out.reshape(B, M, N)
